# baseline (speedup 1.0000x reference)
import functools

import jax
import jax.numpy as jnp
from jax import lax
from jax.experimental import pallas as pl
from jax.experimental.pallas import tpu as pltpu
from jax.experimental.pallas import tpu_sc as plsc

_NC = 2
_NS = 16
_NW = _NC * _NS
_CHUNK = 128


@functools.lru_cache(maxsize=None)
def _build(n, d):
    per_w = n // _NW
    nch = per_w // _CHUNK
    mesh = plsc.VectorSubcoreMesh(core_axis_name="c", subcore_axis_name="s")

    k_grp = 1
    big = 640
    group = k_grp * big
    ng = per_w // group

    @functools.partial(
        pl.kernel,
        out_type=jax.ShapeDtypeStruct((n, d), jnp.float32),
        mesh=mesh,
        compiler_params=pltpu.CompilerParams(use_tc_tiling_on_sc=False),
        scratch_types=[
            pltpu.VMEM((per_w,), jnp.int32),
            pltpu.VMEM((group, d), jnp.float32),
            pltpu.VMEM((group, d), jnp.float32),
            pltpu.SemaphoreType.DMA,
            pltpu.SemaphoreType.DMA,
            pltpu.SemaphoreType.DMA,
            pltpu.SemaphoreType.DMA,
        ],
    )
    def grab(idx_hbm, table_hbm, out_hbm, idx_v, rows0, rows1,
             gsem0, gsem1, wsem0, wsem1):
        wid = lax.axis_index("s") * _NC + lax.axis_index("c")
        base = wid * per_w
        pltpu.sync_copy(idx_hbm.at[pl.ds(base, per_w)], idx_v)

        rows = (rows0, rows1)
        gsem = (gsem0, gsem1)
        wsem = (wsem0, wsem1)

        def slot(g, b, first, wb=True):
            if not first and wb:
                pltpu.make_async_copy(
                    rows[b], out_hbm.at[pl.ds(base, group)], wsem[b]
                ).wait()
            descs = []
            for c in range(k_grp):
                start = g * group + c * big
                descs.append(
                    pltpu.async_copy(
                        table_hbm.at[idx_v.at[pl.ds(start, big)]],
                        rows[b].at[pl.ds(c * big, big)],
                        gsem[b],
                    )
                )
            for desc in descs:
                desc.wait()
            if wb:
                pltpu.async_copy(
                    rows[b], out_hbm.at[pl.ds(base + g * group, group)], wsem[b]
                )

        slot(0, 0, True)
        slot(1, 1, True)

        def body(g2, carry):
            slot(2 * g2, 0, False)
            slot(2 * g2 + 1, 1, False)
            return carry

        lax.fori_loop(1, ng // 2, body, 0)

        pltpu.make_async_copy(
            rows0, out_hbm.at[pl.ds(base, group)], wsem0
        ).wait()
        pltpu.make_async_copy(
            rows1, out_hbm.at[pl.ds(base, group)], wsem1
        ).wait()

    return grab


@jax.jit
def kernel(x, table):
    b, h = x.shape
    d = table.shape[1]
    idx = x.reshape(-1).astype(jnp.int32)
    out = _build(b * h, d)(idx, table)
    zero = table[0, 0] * 0.0
    return out.reshape(b, h, d) + zero


# speedup vs baseline: 2.4674x; 2.4674x over previous
import functools

import jax
import jax.numpy as jnp
from jax import lax
from jax.experimental import pallas as pl
from jax.experimental.pallas import tpu as pltpu
from jax.experimental.pallas import tpu_sc as plsc

_NC = 2
_NS = 16
_NW = _NC * _NS
_CHUNK = 128


@functools.lru_cache(maxsize=None)
def _build(n, d, n_out_rows):
    per_w = n // _NW
    nch = per_w // _CHUNK
    k_grp = 5
    group = k_grp * _CHUNK
    ng = per_w // group
    mesh = plsc.VectorSubcoreMesh(core_axis_name="c", subcore_axis_name="s")

    @functools.partial(
        pl.kernel,
        out_type=jax.ShapeDtypeStruct((n_out_rows, d), jnp.float32),
        mesh=mesh,
        compiler_params=pltpu.CompilerParams(use_tc_tiling_on_sc=False),
        scratch_types=[
            pltpu.VMEM((per_w,), jnp.int32),
            pltpu.VMEM((nch, _CHUNK), jnp.int32),
            pltpu.VMEM((group, d), jnp.float32),
            pltpu.VMEM((group, d), jnp.float32),
            pltpu.SemaphoreType.DMA,
            pltpu.SemaphoreType.DMA,
            pltpu.SemaphoreType.DMA,
            pltpu.SemaphoreType.DMA,
        ],
    )
    def grab(idx_hbm, sidx_hbm, table_hbm, out_hbm, idx_v, sidx_v,
             rows0, rows1, gsem0, gsem1, wsem0, wsem1):
        wid = lax.axis_index("s") * _NC + lax.axis_index("c")
        base = wid * per_w
        pltpu.sync_copy(idx_hbm.at[pl.ds(base, per_w)], idx_v)
        pltpu.sync_copy(sidx_hbm.at[pl.ds(wid * nch, nch)], sidx_v)

        rows = (rows0, rows1)
        gsem = (gsem0, gsem1)
        wsem = (wsem0, wsem1)

        def slot(g, b, first):
            if not first:
                pltpu.make_async_copy(
                    rows[b], out_hbm.at[pl.ds(0, group)], wsem[b]
                ).wait()
            descs = []
            for c in range(k_grp):
                start = g * group + c * _CHUNK
                descs.append(
                    pltpu.async_copy(
                        table_hbm.at[idx_v.at[pl.ds(start, _CHUNK)]],
                        rows[b].at[pl.ds(c * _CHUNK, _CHUNK)],
                        gsem[b],
                    )
                )
            for desc in descs:
                desc.wait()
            for c in range(k_grp):
                pltpu.async_copy(
                    rows[b].at[pl.ds(c * _CHUNK, _CHUNK)],
                    out_hbm.at[sidx_v.at[g * k_grp + c]],
                    wsem[b],
                )

        slot(0, 0, True)
        slot(1, 1, True)

        def body(g2, carry):
            slot(2 * g2, 0, False)
            slot(2 * g2 + 1, 1, False)
            return carry

        lax.fori_loop(1, ng // 2, body, 0)

        pltpu.make_async_copy(
            rows0, out_hbm.at[pl.ds(0, group)], wsem0
        ).wait()
        pltpu.make_async_copy(
            rows1, out_hbm.at[pl.ds(0, group)], wsem1
        ).wait()

    return grab


@jax.jit
def kernel(x, table):
    b, h = x.shape
    d = table.shape[1]
    n = b * h
    hpad = ((h + 7) // 8) * 8
    dpad = 128
    sub = dpad // d
    idx = x.reshape(-1).astype(jnp.int32)
    r = jnp.arange(n, dtype=jnp.int32)
    srow = sub * ((r // h) * hpad + (r % h))
    out = _build(n, d, b * hpad * sub)(idx, srow.reshape(-1, _CHUNK), table)
    return out.reshape(b, hpad, dpad)[:, :h, :d]


# speedup vs baseline: 2.4678x; 1.0002x over previous
import functools

import jax
import jax.numpy as jnp
from jax import lax
from jax.experimental import pallas as pl
from jax.experimental.pallas import tpu as pltpu
from jax.experimental.pallas import tpu_sc as plsc

_NC = 2
_NS = 16
_NW = _NC * _NS
_CHUNK = 128


@functools.lru_cache(maxsize=None)
def _build(n, d, n_out_rows):
    per_w = n // _NW
    nch = per_w // _CHUNK
    k_grp = 5
    group = k_grp * _CHUNK
    ng = per_w // group
    mesh = plsc.VectorSubcoreMesh(core_axis_name="c", subcore_axis_name="s")

    @functools.partial(
        pl.kernel,
        out_type=jax.ShapeDtypeStruct((n_out_rows, d), jnp.float32),
        mesh=mesh,
        compiler_params=pltpu.CompilerParams(use_tc_tiling_on_sc=False),
        scratch_types=[
            pltpu.VMEM((2 * nch, _CHUNK), jnp.int32),
            pltpu.VMEM((group, d), jnp.float32),
            pltpu.VMEM((group, d), jnp.float32),
            pltpu.SemaphoreType.DMA,
            pltpu.SemaphoreType.DMA,
            pltpu.SemaphoreType.DMA,
        ],
    )
    def grab(meta_hbm, table_hbm, out_hbm, meta_v,
             rows0, rows1, gsem, wsem0, wsem1):
        wid = lax.axis_index("s") * _NC + lax.axis_index("c")
        pltpu.sync_copy(meta_hbm.at[pl.ds(wid * 2 * nch, 2 * nch)], meta_v)

        rows = (rows0, rows1)
        wsem = (wsem0, wsem1)

        def slot(g, b, first):
            if not first:
                pltpu.make_async_copy(
                    rows[b], out_hbm.at[pl.ds(0, group)], wsem[b]
                ).wait()
            descs = []
            for c in range(k_grp):
                descs.append(
                    pltpu.async_copy(
                        table_hbm.at[meta_v.at[g * k_grp + c]],
                        rows[b].at[pl.ds(c * _CHUNK, _CHUNK)],
                        gsem,
                    )
                )
            for desc in descs:
                desc.wait()
            for c in range(k_grp):
                pltpu.async_copy(
                    rows[b].at[pl.ds(c * _CHUNK, _CHUNK)],
                    out_hbm.at[meta_v.at[nch + g * k_grp + c]],
                    wsem[b],
                )

        slot(0, 0, True)
        slot(1, 1, True)

        def body(g2, carry):
            slot(2 * g2, 0, False)
            slot(2 * g2 + 1, 1, False)
            return carry

        lax.fori_loop(1, ng // 2, body, 0)

        pltpu.make_async_copy(
            rows0, out_hbm.at[pl.ds(0, group)], wsem0
        ).wait()
        pltpu.make_async_copy(
            rows1, out_hbm.at[pl.ds(0, group)], wsem1
        ).wait()

    return grab


@jax.jit
def kernel(x, table):
    b, h = x.shape
    d = table.shape[1]
    n = b * h
    hpad = ((h + 7) // 8) * 8
    dpad = 128
    sub = dpad // d
    per_w = n // _NW
    nch = per_w // _CHUNK
    idx = x.reshape(-1).astype(jnp.int32)
    r = jnp.arange(n, dtype=jnp.int32)
    srow = sub * ((r // h) * hpad + (r % h))
    meta = jnp.concatenate(
        [idx.reshape(_NW, nch, _CHUNK), srow.reshape(_NW, nch, _CHUNK)],
        axis=1,
    ).reshape(-1, _CHUNK)
    out = _build(n, d, b * hpad * sub)(meta, table)
    return out.reshape(b, hpad, dpad)[:, :h, :d]
